# Initial kernel scaffold; baseline (speedup 1.0000x reference)
import jax
import jax.numpy as jnp
from jax.experimental import pallas as pl


def kernel(grid_node_features, edge_index):
    raise NotImplementedError("write your pallas kernel here")


# speedup vs baseline: 81.0226x; 81.0226x over previous
import functools

import jax
import jax.numpy as jnp
from jax import lax
from jax.experimental import pallas as pl
from jax.experimental.pallas import tpu as pltpu
from jax.experimental.pallas import tpu_sc as plsc

B = 2
G = 10000
M = 2500
E = 320000
D = 128

NC = 2
NS = 16
NW = NC * NS
EPW = E // NW
CH = 100
NCHUNK = EPW // CH
W = B * D + 16
MPAD = 2560
RPS = MPAD // NS


def _sc_scatter(table, src3, dst3):
  mesh = plsc.VectorSubcoreMesh(core_axis_name="c", subcore_axis_name="s")

  @functools.partial(
      pl.kernel,
      mesh=mesh,
      out_type=jax.ShapeDtypeStruct((NC, MPAD, W), jnp.float32),
      compiler_params=pltpu.CompilerParams(use_tc_tiling_on_sc=False),
      scratch_types=[
          pltpu.VMEM((NCHUNK, CH), jnp.int32),
          pltpu.VMEM((NCHUNK, CH), jnp.int32),
          pltpu.VMEM((CH, W), jnp.float32),
          pltpu.VMEM((CH, W), jnp.float32),
          pltpu.VMEM_SHARED((MPAD, W), jnp.float32),
          pltpu.SemaphoreType.DMA,
          pltpu.SemaphoreType.DMA,
      ],
  )
  def k(table_hbm, src_hbm, dst_hbm, out_hbm,
        src_v, dst_v, rows0, rows1, acc, sem0, sem1):
    c = lax.axis_index("c")
    s = lax.axis_index("s")
    w = c * NS + s

    pltpu.sync_copy(src_hbm.at[w], src_v)
    pltpu.sync_copy(dst_hbm.at[w], dst_v)

    def zrow(r, carry):
      def zcol(kk, inner):
        rows0[r, pl.ds(kk * 16, 16)] = jnp.zeros((16,), jnp.float32)
        return inner
      return lax.fori_loop(0, W // 16, zcol, carry)
    lax.fori_loop(0, CH, zrow, 0)
    pltpu.sync_copy(rows0.at[pl.ds(0, RPS // 2)], acc.at[pl.ds(s * RPS, RPS // 2)])
    pltpu.sync_copy(rows0.at[pl.ds(0, RPS // 2)],
                    acc.at[pl.ds(s * RPS + RPS // 2, RPS // 2)])
    plsc.subcore_barrier()

    def gather_start(j, buf, sem):
      pltpu.async_copy(table_hbm.at[src_v.at[j]], buf, sem)

    def gather_wait(j, buf, sem):
      pltpu.make_async_copy(table_hbm.at[src_v.at[j]], buf, sem).wait()

    def scatter_add(j, buf):
      pltpu.sync_copy(buf, acc.at[dst_v.at[j]], add=True)

    gather_start(0, rows0, sem0)

    def body(i, carry):
      j = i * 2
      gather_start(j + 1, rows1, sem1)
      gather_wait(j, rows0, sem0)
      scatter_add(j, rows0)
      gather_start(j + 2, rows0, sem0)
      gather_wait(j + 1, rows1, sem1)
      scatter_add(j + 1, rows1)
      return carry
    lax.fori_loop(0, NCHUNK // 2 - 1, body, 0)

    j = NCHUNK - 2
    gather_start(j + 1, rows1, sem1)
    gather_wait(j, rows0, sem0)
    scatter_add(j, rows0)
    gather_wait(j + 1, rows1, sem1)
    scatter_add(j + 1, rows1)

    plsc.subcore_barrier()
    h = RPS // 2
    pltpu.sync_copy(acc.at[pl.ds(s * RPS, h)], rows0.at[pl.ds(0, h)])
    pltpu.sync_copy(rows0.at[pl.ds(0, h)], out_hbm.at[c, pl.ds(s * RPS, h)])
    pltpu.sync_copy(acc.at[pl.ds(s * RPS + h, h)], rows1.at[pl.ds(0, h)])
    pltpu.sync_copy(rows1.at[pl.ds(0, h)], out_hbm.at[c, pl.ds(s * RPS + h, h)])

  return k(table, src3, dst3)


def _combine(acc):
  def body(acc_ref, out_ref):
    ssum = acc_ref[0] + acc_ref[1]
    cnt = jnp.maximum(ssum[:, B * D:B * D + 1], 1.0)
    out_ref[0] = ssum[:, :D] / cnt
    out_ref[1] = ssum[:, D:B * D] / cnt

  return pl.pallas_call(
      body,
      out_shape=jax.ShapeDtypeStruct((B, MPAD, D), jnp.float32),
  )(acc)


def kernel(grid_node_features, edge_index):
  src = edge_index[:, 0].astype(jnp.int32).reshape(NW, NCHUNK, CH)
  dst = edge_index[:, 1].astype(jnp.int32).reshape(NW, NCHUNK, CH)
  g2 = jnp.transpose(grid_node_features, (1, 0, 2)).reshape(G, B * D)
  table = jnp.concatenate(
      [g2, jnp.ones((G, 1), jnp.float32), jnp.zeros((G, W - B * D - 1), jnp.float32)],
      axis=1)
  acc = _sc_scatter(table, src, dst)
  out = _combine(acc)
  return out[:, :M]
